# baseline (device time: 32401 ns/iter reference)
import jax
import jax.numpy as jnp
from jax import lax
from jax.experimental import pallas as pl
from jax.experimental.pallas import tpu as pltpu

N_DEV = 32
B, SQ, D_MODEL = 2, 128, 512
HQ_LOCAL, DH = 4, 64
HD_LOCAL = HQ_LOCAL * DH
ROWS = B * SQ
CHUNK = ROWS // N_DEV


def kernel(x, Wq, K_ext, V_ext, Wo):
    me_out = lax.axis_index("i")
    Wq_s = lax.dynamic_slice(Wq, (0, me_out * HD_LOCAL), (D_MODEL, HD_LOCAL))
    x2 = x.reshape(ROWS, D_MODEL)

    def body(x_ref, wq_ref, k_ref, v_ref, wo_ref, out_ref,
             wo_vs, pbf_ref, rs_ref, red_ref, ag_ref,
             lsem, ssem1, rsem1, ssem2, rsem2):
        me = lax.axis_index("i")

        cp_wo = pltpu.make_async_copy(
            wo_ref.at[pl.ds(me * HD_LOCAL, HD_LOCAL), :], wo_vs, lsem)
        cp_wo.start()

        barrier = pltpu.get_barrier_semaphore()
        for t in range(1, N_DEV):
            j = lax.rem(me + t, N_DEV)
            pl.semaphore_signal(
                barrier, inc=1,
                device_id=(j,), device_id_type=pl.DeviceIdType.MESH,
            )

        Q = jnp.dot(x_ref[...].astype(jnp.bfloat16),
                    wq_ref[...].astype(jnp.bfloat16),
                    preferred_element_type=jnp.float32)
        brows = []
        for b in range(B):
            heads = []
            for h in range(HQ_LOCAL):
                q = Q[b * SQ:(b + 1) * SQ,
                      h * DH:(h + 1) * DH].astype(jnp.bfloat16)
                k = k_ref[b, :, h, :].astype(jnp.bfloat16)
                v = v_ref[b, :, h, :].astype(jnp.bfloat16)
                s = lax.dot_general(
                    q, k, (((1,), (1,)), ((), ())),
                    preferred_element_type=jnp.float32) * 0.125
                m = jnp.max(s, axis=1, keepdims=True)
                w = jnp.exp(s - m)
                w = (w / jnp.sum(w, axis=1, keepdims=True)).astype(
                    jnp.bfloat16)
                heads.append(jnp.dot(w, v,
                                     preferred_element_type=jnp.float32))
            brows.append(jnp.concatenate(heads, axis=1))
        ctx = jnp.concatenate(brows, axis=0).astype(jnp.bfloat16)
        cp_wo.wait()
        part = jnp.dot(ctx, wo_vs[...].astype(jnp.bfloat16),
                       preferred_element_type=jnp.float32)
        pbf_ref[...] = part.astype(jnp.bfloat16).reshape(
            N_DEV, CHUNK, D_MODEL)

        rs_ref[pl.ds(me, 1)] = pbf_ref[pl.ds(me, 1)]

        pl.semaphore_wait(barrier, N_DEV - 1)

        sends1 = []
        for t in range(1, N_DEV):
            j = lax.rem(me + t, N_DEV)
            r = pltpu.make_async_remote_copy(
                src_ref=pbf_ref.at[j],
                dst_ref=rs_ref.at[me],
                send_sem=ssem1.at[t - 1],
                recv_sem=rsem1.at[me],
                device_id=(j,),
                device_id_type=pl.DeviceIdType.MESH,
            )
            r.start()
            sends1.append(r)
        for t in range(1, N_DEV):
            j = lax.rem(me + t, N_DEV)
            rr = pltpu.make_async_remote_copy(
                src_ref=pbf_ref.at[0],
                dst_ref=rs_ref.at[j],
                send_sem=ssem1.at[t - 1],
                recv_sem=rsem1.at[j],
                device_id=(j,),
                device_id_type=pl.DeviceIdType.MESH,
            )
            rr.wait_recv()

        red = jnp.sum(rs_ref[...].astype(jnp.float32), axis=0)
        red_ref[...] = red.astype(jnp.bfloat16)

        sends2 = []
        for t in range(1, N_DEV):
            j = lax.rem(me + t, N_DEV)
            r = pltpu.make_async_remote_copy(
                src_ref=red_ref,
                dst_ref=ag_ref.at[me],
                send_sem=ssem2.at[t - 1],
                recv_sem=rsem2.at[me],
                device_id=(j,),
                device_id_type=pl.DeviceIdType.MESH,
            )
            r.start()
            sends2.append(r)
        for t in range(1, N_DEV):
            j = lax.rem(me + t, N_DEV)
            rr = pltpu.make_async_remote_copy(
                src_ref=red_ref,
                dst_ref=ag_ref.at[j],
                send_sem=ssem2.at[t - 1],
                recv_sem=rsem2.at[j],
                device_id=(j,),
                device_id_type=pl.DeviceIdType.MESH,
            )
            rr.wait_recv()

        out_ref[...] = ag_ref[...].astype(jnp.float32).reshape(
            ROWS, D_MODEL)
        out_ref[pl.ds(me * CHUNK, CHUNK)] = red

        for r in sends1:
            r.wait_send()
        for r in sends2:
            r.wait_send()

    out2 = pl.pallas_call(
        body,
        out_shape=jax.ShapeDtypeStruct((ROWS, D_MODEL), jnp.float32),
        in_specs=[
            pl.BlockSpec(memory_space=pltpu.VMEM),
            pl.BlockSpec(memory_space=pltpu.VMEM),
            pl.BlockSpec(memory_space=pltpu.VMEM),
            pl.BlockSpec(memory_space=pltpu.VMEM),
            pl.BlockSpec(memory_space=pltpu.MemorySpace.HBM),
        ],
        out_specs=pl.BlockSpec(memory_space=pltpu.VMEM),
        scratch_shapes=[
            pltpu.VMEM((HD_LOCAL, D_MODEL), jnp.float32),
            pltpu.VMEM((N_DEV, CHUNK, D_MODEL), jnp.bfloat16),
            pltpu.VMEM((N_DEV, CHUNK, D_MODEL), jnp.bfloat16),
            pltpu.VMEM((CHUNK, D_MODEL), jnp.bfloat16),
            pltpu.VMEM((N_DEV, CHUNK, D_MODEL), jnp.bfloat16),
            pltpu.SemaphoreType.DMA,
            pltpu.SemaphoreType.DMA((N_DEV - 1,)),
            pltpu.SemaphoreType.DMA((N_DEV,)),
            pltpu.SemaphoreType.DMA((N_DEV - 1,)),
            pltpu.SemaphoreType.DMA((N_DEV,)),
        ],
        compiler_params=pltpu.CompilerParams(collective_id=0),
    )(x2, Wq_s, K_ext, V_ext, Wo)
    return out2.reshape(B, SQ, D_MODEL)


# device time: 28394 ns/iter; 1.1411x vs baseline; 1.1411x over previous
import jax
import jax.numpy as jnp
from jax import lax
from jax.experimental import pallas as pl
from jax.experimental.pallas import tpu as pltpu

N_DEV = 32
B, SQ, D_MODEL = 2, 128, 512
HQ_LOCAL, DH = 4, 64
HD_LOCAL = HQ_LOCAL * DH
ROWS = B * SQ
GCHUNK = SQ // N_DEV


def kernel(x, Wq, K_ext, V_ext, Wo):
    me_out = lax.axis_index("i")
    Wq_s = lax.dynamic_slice(Wq, (0, me_out * HD_LOCAL), (D_MODEL, HD_LOCAL))
    Wo_s = lax.dynamic_slice(Wo, (me_out * HD_LOCAL, 0), (HD_LOCAL, D_MODEL))
    x2 = x.reshape(ROWS, D_MODEL)

    def body(x_ref, wq_ref, k_ref, v_ref, wo_ref, out_ref,
             pbf0, pbf1, rs0, rs1, red0_ref, red1_ref, ag0, ag1,
             ssem1a, rsem1a, ssem1b, rsem1b,
             ssem2a, rsem2a, ssem2b, rsem2b):
        me = lax.axis_index("i")

        barrier = pltpu.get_barrier_semaphore()
        for t in range(1, N_DEV):
            j = lax.rem(me + t, N_DEV)
            pl.semaphore_signal(
                barrier, inc=1,
                device_id=(j,), device_id_type=pl.DeviceIdType.MESH,
            )

        Q = jnp.dot(x_ref[...].astype(jnp.bfloat16),
                    wq_ref[...].astype(jnp.bfloat16),
                    preferred_element_type=jnp.float32)
        wo_b = wo_ref[...].astype(jnp.bfloat16)

        def attn_batch(b):
            heads = []
            for h in range(HQ_LOCAL):
                q = Q[b * SQ:(b + 1) * SQ,
                      h * DH:(h + 1) * DH].astype(jnp.bfloat16)
                k = k_ref[b, :, h, :].astype(jnp.bfloat16)
                v = v_ref[b, :, h, :].astype(jnp.bfloat16)
                s = lax.dot_general(
                    q, k, (((1,), (1,)), ((), ())),
                    preferred_element_type=jnp.float32) * 0.125
                m = jnp.max(s, axis=1, keepdims=True)
                w = jnp.exp(s - m)
                w = (w / jnp.sum(w, axis=1, keepdims=True)).astype(
                    jnp.bfloat16)
                heads.append(jnp.dot(w, v,
                                     preferred_element_type=jnp.float32))
            ctx = jnp.concatenate(heads, axis=1).astype(jnp.bfloat16)
            return jnp.dot(ctx, wo_b,
                           preferred_element_type=jnp.float32)

        def rs_send(pbf, ssem, rsem):
            sends = []
            for t in range(1, N_DEV):
                j = lax.rem(me + t, N_DEV)
                r = pltpu.make_async_remote_copy(
                    src_ref=pbf.at[j],
                    dst_ref=(rs0 if pbf is pbf0 else rs1).at[me],
                    send_sem=ssem.at[t - 1],
                    recv_sem=rsem.at[me],
                    device_id=(j,),
                    device_id_type=pl.DeviceIdType.MESH,
                )
                r.start()
                sends.append(r)
            return sends

        def wait_recvs(rs, pbf, ssem, rsem):
            for t in range(1, N_DEV):
                j = lax.rem(me + t, N_DEV)
                rr = pltpu.make_async_remote_copy(
                    src_ref=pbf.at[0],
                    dst_ref=rs.at[j],
                    send_sem=ssem.at[t - 1],
                    recv_sem=rsem.at[j],
                    device_id=(j,),
                    device_id_type=pl.DeviceIdType.MESH,
                )
                rr.wait_recv()

        def ag_send(red_ref, ag, ssem, rsem):
            sends = []
            for t in range(1, N_DEV):
                j = lax.rem(me + t, N_DEV)
                r = pltpu.make_async_remote_copy(
                    src_ref=red_ref,
                    dst_ref=ag.at[me],
                    send_sem=ssem.at[t - 1],
                    recv_sem=rsem.at[me],
                    device_id=(j,),
                    device_id_type=pl.DeviceIdType.MESH,
                )
                r.start()
                sends.append(r)
            return sends

        part0 = attn_batch(0)
        pbf0[...] = part0.astype(jnp.bfloat16).reshape(
            N_DEV, GCHUNK, D_MODEL)
        rs0[pl.ds(me, 1)] = pbf0[pl.ds(me, 1)]
        pl.semaphore_wait(barrier, N_DEV - 1)
        sends1a = rs_send(pbf0, ssem1a, rsem1a)

        part1 = attn_batch(1)
        pbf1[...] = part1.astype(jnp.bfloat16).reshape(
            N_DEV, GCHUNK, D_MODEL)
        rs1[pl.ds(me, 1)] = pbf1[pl.ds(me, 1)]
        sends1b = rs_send(pbf1, ssem1b, rsem1b)

        wait_recvs(rs0, pbf0, ssem1a, rsem1a)
        red0_ref[...] = jnp.sum(rs0[...].astype(jnp.float32),
                                axis=0).astype(jnp.bfloat16)
        ag0[pl.ds(me, 1)] = red0_ref[...].reshape(1, GCHUNK, D_MODEL)
        sends2a = ag_send(red0_ref, ag0, ssem2a, rsem2a)

        wait_recvs(rs1, pbf1, ssem1b, rsem1b)
        red1_ref[...] = jnp.sum(rs1[...].astype(jnp.float32),
                                axis=0).astype(jnp.bfloat16)
        ag1[pl.ds(me, 1)] = red1_ref[...].reshape(1, GCHUNK, D_MODEL)
        sends2b = ag_send(red1_ref, ag1, ssem2b, rsem2b)

        wait_recvs(ag0, pbf0, ssem2a, rsem2a)
        out_ref[pl.ds(0, SQ)] = ag0[...].astype(jnp.float32).reshape(
            SQ, D_MODEL)
        wait_recvs(ag1, pbf1, ssem2b, rsem2b)
        out_ref[pl.ds(SQ, SQ)] = ag1[...].astype(jnp.float32).reshape(
            SQ, D_MODEL)

        for r in sends1a + sends1b + sends2a + sends2b:
            r.wait_send()

    gshape = (N_DEV, GCHUNK, D_MODEL)
    out2 = pl.pallas_call(
        body,
        out_shape=jax.ShapeDtypeStruct((ROWS, D_MODEL), jnp.float32),
        in_specs=[pl.BlockSpec(memory_space=pltpu.VMEM)] * 5,
        out_specs=pl.BlockSpec(memory_space=pltpu.VMEM),
        scratch_shapes=[
            pltpu.VMEM(gshape, jnp.bfloat16),
            pltpu.VMEM(gshape, jnp.bfloat16),
            pltpu.VMEM(gshape, jnp.bfloat16),
            pltpu.VMEM(gshape, jnp.bfloat16),
            pltpu.VMEM((GCHUNK, D_MODEL), jnp.bfloat16),
            pltpu.VMEM((GCHUNK, D_MODEL), jnp.bfloat16),
            pltpu.VMEM(gshape, jnp.bfloat16),
            pltpu.VMEM(gshape, jnp.bfloat16),
            pltpu.SemaphoreType.DMA((N_DEV - 1,)),
            pltpu.SemaphoreType.DMA((N_DEV,)),
            pltpu.SemaphoreType.DMA((N_DEV - 1,)),
            pltpu.SemaphoreType.DMA((N_DEV,)),
            pltpu.SemaphoreType.DMA((N_DEV - 1,)),
            pltpu.SemaphoreType.DMA((N_DEV,)),
            pltpu.SemaphoreType.DMA((N_DEV - 1,)),
            pltpu.SemaphoreType.DMA((N_DEV,)),
        ],
        compiler_params=pltpu.CompilerParams(collective_id=0),
    )(x2, Wq_s, K_ext, V_ext, Wo_s)
    return out2.reshape(B, SQ, D_MODEL)


# device time: 26407 ns/iter; 1.2270x vs baseline; 1.0752x over previous
import jax
import jax.numpy as jnp
from jax import lax
from jax.experimental import pallas as pl
from jax.experimental.pallas import tpu as pltpu

N_DEV = 32
B, SQ, D_MODEL = 2, 128, 512
HQ_LOCAL, DH = 4, 64
HD_LOCAL = HQ_LOCAL * DH
ROWS = B * SQ
CHUNK = ROWS // N_DEV


def kernel(x, Wq, K_ext, V_ext, Wo):
    me_out = lax.axis_index("i")
    Wq_s = lax.dynamic_slice(Wq, (0, me_out * HD_LOCAL), (D_MODEL, HD_LOCAL))
    Wo_s = lax.dynamic_slice(Wo, (me_out * HD_LOCAL, 0), (HD_LOCAL, D_MODEL))
    x2 = x.reshape(ROWS, D_MODEL)
    K2 = K_ext.reshape(B, SQ, HD_LOCAL)
    V2 = V_ext.reshape(B, SQ, HD_LOCAL)

    def body(x_ref, wq_ref, k_ref, v_ref, wo_ref, out_ref,
             pbf_ref, rs_ref, red_ref, ag_ref, ssem1, rsem1, ssem2, rsem2):
        me = lax.axis_index("i")

        barrier = pltpu.get_barrier_semaphore()
        for t in range(1, N_DEV):
            j = lax.rem(me + t, N_DEV)
            pl.semaphore_signal(
                barrier, inc=1,
                device_id=(j,), device_id_type=pl.DeviceIdType.MESH,
            )

        Q = jnp.dot(x_ref[...].astype(jnp.bfloat16),
                    wq_ref[...].astype(jnp.bfloat16),
                    preferred_element_type=jnp.float32)
        brows = []
        for b in range(B):
            heads = []
            for h in range(HQ_LOCAL):
                q = Q[b * SQ:(b + 1) * SQ,
                      h * DH:(h + 1) * DH].astype(jnp.bfloat16)
                k = k_ref[b, :, h * DH:(h + 1) * DH].astype(jnp.bfloat16)
                v = v_ref[b, :, h * DH:(h + 1) * DH].astype(jnp.bfloat16)
                s = lax.dot_general(
                    q, k, (((1,), (1,)), ((), ())),
                    preferred_element_type=jnp.float32) * 0.125
                m = jnp.max(s, axis=1, keepdims=True)
                w = jnp.exp(s - m)
                w = (w / jnp.sum(w, axis=1, keepdims=True)).astype(
                    jnp.bfloat16)
                heads.append(jnp.dot(w, v,
                                     preferred_element_type=jnp.float32))
            brows.append(jnp.concatenate(heads, axis=1))
        ctx = jnp.concatenate(brows, axis=0).astype(jnp.bfloat16)
        part = jnp.dot(ctx, wo_ref[...].astype(jnp.bfloat16),
                       preferred_element_type=jnp.float32)
        pbf_ref[...] = part.astype(jnp.bfloat16).reshape(
            N_DEV, CHUNK, D_MODEL)

        rs_ref[pl.ds(me, 1)] = pbf_ref[pl.ds(me, 1)]

        pl.semaphore_wait(barrier, N_DEV - 1)

        sends1 = []
        for t in range(1, N_DEV):
            j = lax.rem(me + t, N_DEV)
            r = pltpu.make_async_remote_copy(
                src_ref=pbf_ref.at[j],
                dst_ref=rs_ref.at[me],
                send_sem=ssem1.at[t - 1],
                recv_sem=rsem1.at[me],
                device_id=(j,),
                device_id_type=pl.DeviceIdType.MESH,
            )
            r.start()
            sends1.append(r)
        for t in range(1, N_DEV):
            j = lax.rem(me + t, N_DEV)
            rr = pltpu.make_async_remote_copy(
                src_ref=pbf_ref.at[0],
                dst_ref=rs_ref.at[j],
                send_sem=ssem1.at[t - 1],
                recv_sem=rsem1.at[j],
                device_id=(j,),
                device_id_type=pl.DeviceIdType.MESH,
            )
            rr.wait_recv()

        red = jnp.sum(rs_ref[...].astype(jnp.float32), axis=0)
        red_ref[...] = red.astype(jnp.bfloat16)

        sends2 = []
        for t in range(1, N_DEV):
            j = lax.rem(me + t, N_DEV)
            r = pltpu.make_async_remote_copy(
                src_ref=red_ref,
                dst_ref=ag_ref.at[me],
                send_sem=ssem2.at[t - 1],
                recv_sem=rsem2.at[me],
                device_id=(j,),
                device_id_type=pl.DeviceIdType.MESH,
            )
            r.start()
            sends2.append(r)
        for t in range(1, N_DEV):
            j = lax.rem(me + t, N_DEV)
            rr = pltpu.make_async_remote_copy(
                src_ref=red_ref,
                dst_ref=ag_ref.at[j],
                send_sem=ssem2.at[t - 1],
                recv_sem=rsem2.at[j],
                device_id=(j,),
                device_id_type=pl.DeviceIdType.MESH,
            )
            rr.wait_recv()

        out_ref[...] = ag_ref[...].astype(jnp.float32).reshape(
            B, SQ, D_MODEL)
        b_own = me // (SQ // CHUNK)
        row_off = me * CHUNK - b_own * SQ
        out_ref[pl.ds(b_own, 1), pl.ds(row_off, CHUNK), :] = red.reshape(
            1, CHUNK, D_MODEL)

        for r in sends1:
            r.wait_send()
        for r in sends2:
            r.wait_send()

    out2 = pl.pallas_call(
        body,
        out_shape=jax.ShapeDtypeStruct((B, SQ, D_MODEL), jnp.float32),
        in_specs=[pl.BlockSpec(memory_space=pltpu.VMEM)] * 5,
        out_specs=pl.BlockSpec(memory_space=pltpu.VMEM),
        scratch_shapes=[
            pltpu.VMEM((N_DEV, CHUNK, D_MODEL), jnp.bfloat16),
            pltpu.VMEM((N_DEV, CHUNK, D_MODEL), jnp.bfloat16),
            pltpu.VMEM((CHUNK, D_MODEL), jnp.bfloat16),
            pltpu.VMEM((N_DEV, CHUNK, D_MODEL), jnp.bfloat16),
            pltpu.SemaphoreType.DMA((N_DEV - 1,)),
            pltpu.SemaphoreType.DMA((N_DEV,)),
            pltpu.SemaphoreType.DMA((N_DEV - 1,)),
            pltpu.SemaphoreType.DMA((N_DEV,)),
        ],
        compiler_params=pltpu.CompilerParams(collective_id=0),
    )(x2, Wq_s, K2, V2, Wo_s)
    return out2
